# baseline (speedup 1.0000x reference)
import jax
import jax.numpy as jnp
from jax import lax
from jax.experimental import pallas as pl
from jax.experimental.pallas import tpu as pltpu
from jax.experimental.pallas import tpu_sc as plsc

N = 10000
E = 320000
F = 128
R = 64
CHUNK = 64
NSUB = 16
NB = 320
EPAD = NSUB * NB * CHUNK - E
IB = 64
NIB = NB // IB
NBUF = 4
ROWS_PT = N // NSUB
NA = N + 8
DH = 16

_mesh = plsc.VectorSubcoreMesh(core_axis_name="c", subcore_axis_name="s")
_sc_params = pltpu.CompilerParams(use_tc_tiling_on_sc=False)


def _deg_body(srcd3, dstd3, zhist, onesb, degs, degd, hist, idxb, onesv, sem):
    del sem
    c = lax.axis_index("c")
    s = lax.axis_index("s")
    row0 = s * ROWS_PT

    pltpu.sync_copy(zhist, hist.at[pl.ds(row0, ROWS_PT)])
    pltpu.sync_copy(onesb, onesv)
    plsc.subcore_barrier()

    def run(idx3, out):
        def outer(b, carry):
            pltpu.sync_copy(idx3.at[s, pl.ds(b * IB, IB)], idxb)

            def body(k, carry2):
                pltpu.sync_copy(onesv, hist.at[idxb.at[k]], add=True)
                return carry2

            return lax.fori_loop(0, IB, body, carry)

        lax.fori_loop(0, NIB, outer, 0)
        plsc.subcore_barrier()
        pltpu.sync_copy(hist.at[pl.ds(row0, ROWS_PT)],
                        out.at[pl.ds(row0, ROWS_PT)])

    pl.when(c == 0)(lambda: run(srcd3, degs))
    pl.when(c == 1)(lambda: run(dstd3, degd))


_deg_call = pl.kernel(
    _deg_body,
    out_type=[jax.ShapeDtypeStruct((N, DH), jnp.float32),
              jax.ShapeDtypeStruct((N, DH), jnp.float32)],
    mesh=_mesh,
    scratch_types=[
        pltpu.VMEM_SHARED((NA, DH), jnp.float32),
        pltpu.VMEM((IB, CHUNK), jnp.int32),
        pltpu.VMEM((CHUNK, DH), jnp.float32),
        pltpu.SemaphoreType.DMA,
    ],
    compiler_params=_sc_params,
)


def _dense_body(feat_ref, deg_ref, w1_ref, w2a_ref, w2b_ref, t0_ref, t1_ref):
    x = feat_ref[...] * lax.rsqrt(jnp.maximum(deg_ref[...], 1.0))
    sfull = jnp.dot(x, w1_ref[...], preferred_element_type=jnp.float32)
    z = jnp.dot(x, w2a_ref[...], preferred_element_type=jnp.float32) + w2b_ref[...]
    t = jnp.tanh(z)
    lp = jnp.log(jnp.maximum(jnp.abs(t), 1e-30))
    sg = (t < 0).astype(jnp.float32)
    t0_ref[...] = jnp.concatenate([sfull[:, :R], lp], axis=1)
    t1_ref[...] = jnp.concatenate([sfull[:, R:], sg], axis=1)


_BLK = 1000

_dense_call = pl.pallas_call(
    _dense_body,
    grid=(N // _BLK,),
    in_specs=[
        pl.BlockSpec((_BLK, F), lambda i: (i, 0)),
        pl.BlockSpec((_BLK, 1), lambda i: (i, 0)),
        pl.BlockSpec((F, F), lambda i: (0, 0)),
        pl.BlockSpec((F, R), lambda i: (0, 0)),
        pl.BlockSpec((1, R), lambda i: (0, 0)),
    ],
    out_specs=[
        pl.BlockSpec((_BLK, F), lambda i: (i, 0)),
        pl.BlockSpec((_BLK, F), lambda i: (i, 0)),
    ],
    out_shape=[jax.ShapeDtypeStruct((N, F), jnp.float32),
               jax.ShapeDtypeStruct((N, F), jnp.float32)],
)


def _agg_body(t0, t1, srca3, dsta3, zrows, acc0, acc1,
              acc, sidxb, didxb, rows, gsems, ssems):
    c = lax.axis_index("c")
    s = lax.axis_index("s")
    row0 = s * ROWS_PT

    pltpu.sync_copy(zrows, acc.at[pl.ds(row0, ROWS_PT)])
    plsc.subcore_barrier()

    def run(tbl, out):
        def outer(b, carry):
            pltpu.sync_copy(srca3.at[s, pl.ds(b * IB, IB)], sidxb)
            pltpu.sync_copy(dsta3.at[s, pl.ds(b * IB, IB)], didxb)
            for j in range(NBUF):
                pltpu.async_copy(tbl.at[sidxb.at[j]], rows[j], gsems[j])

            def body(i, carry2):
                k = NBUF * i
                for j in range(NBUF):
                    pltpu.make_async_copy(
                        tbl.at[sidxb.at[k + j]], rows[j], gsems[j]).wait()
                    pltpu.async_copy(
                        rows[j], acc.at[didxb.at[k + j]], ssems[j], add=True)
                for j in range(NBUF):
                    pltpu.make_async_copy(
                        rows[j], acc.at[didxb.at[k + j]], ssems[j]).wait()

                    def _prefetch(jj=j):
                        pltpu.async_copy(tbl.at[sidxb.at[k + NBUF + jj]],
                                         rows[jj], gsems[jj])

                    pl.when(i < IB // NBUF - 1)(_prefetch)
                return carry2

            return lax.fori_loop(0, IB // NBUF, body, carry)

        lax.fori_loop(0, NIB, outer, 0)
        plsc.subcore_barrier()
        pltpu.sync_copy(acc.at[pl.ds(row0, ROWS_PT)],
                        out.at[pl.ds(row0, ROWS_PT)])

    pl.when(c == 0)(lambda: run(t0, acc0))
    pl.when(c == 1)(lambda: run(t1, acc1))


_agg_call = pl.kernel(
    _agg_body,
    out_type=[jax.ShapeDtypeStruct((N, F), jnp.float32),
              jax.ShapeDtypeStruct((N, F), jnp.float32)],
    mesh=_mesh,
    scratch_types=[
        pltpu.VMEM_SHARED((NA, F), jnp.float32),
        pltpu.VMEM((IB, CHUNK), jnp.int32),
        pltpu.VMEM((IB, CHUNK), jnp.int32),
        [pltpu.VMEM((CHUNK, F), jnp.float32)] * NBUF,
        [pltpu.SemaphoreType.DMA] * NBUF,
        [pltpu.SemaphoreType.DMA] * NBUF,
    ],
    compiler_params=_sc_params,
)


def _final_body(a0_ref, a1_ref, deg_ref, v_ref, out_ref):
    a0 = a0_ref[...]
    a1 = a1_ref[...]
    indeg = deg_ref[...]
    h_sum = jnp.concatenate([a0[:, :R], a1[:, :R]], axis=1)
    lp = a0[:, R:]
    cnt = a1[:, R:]
    sign = 1.0 - 2.0 * (cnt - 2.0 * jnp.floor(cnt * 0.5))
    h_prod = sign * jnp.exp(lp) * (indeg > 0).astype(jnp.float32)
    r = h_sum + jnp.dot(h_prod, v_ref[...], preferred_element_type=jnp.float32)
    out_ref[...] = r * lax.rsqrt(jnp.maximum(indeg, 1.0))


_final_call = pl.pallas_call(
    _final_body,
    grid=(N // _BLK,),
    in_specs=[
        pl.BlockSpec((_BLK, F), lambda i: (i, 0)),
        pl.BlockSpec((_BLK, F), lambda i: (i, 0)),
        pl.BlockSpec((_BLK, 1), lambda i: (i, 0)),
        pl.BlockSpec((R, F), lambda i: (0, 0)),
    ],
    out_specs=pl.BlockSpec((_BLK, F), lambda i: (i, 0)),
    out_shape=jax.ShapeDtypeStruct((N, F), jnp.float32),
)


def kernel(feat, edge_index, w1, w2, v):
    src = edge_index[0]
    dst = edge_index[1]
    padn = jnp.full((EPAD,), N, jnp.int32)
    srcd3 = jnp.concatenate([src, padn]).reshape(NSUB, NB, CHUNK)
    dstd3 = jnp.concatenate([dst, padn]).reshape(NSUB, NB, CHUNK)
    srca3 = jnp.concatenate(
        [src, jnp.zeros((EPAD,), jnp.int32)]).reshape(NSUB, NB, CHUNK)
    zhist = jnp.zeros((ROWS_PT, DH), jnp.float32)
    onesb = jnp.concatenate(
        [jnp.ones((CHUNK, 1), jnp.float32),
         jnp.zeros((CHUNK, DH - 1), jnp.float32)], axis=1)
    zrows = jnp.zeros((ROWS_PT, F), jnp.float32)

    degs, degd = _deg_call(srcd3, dstd3, zhist, onesb)
    outdeg = degs[:, 0:1]
    indeg = degd[:, 0:1]
    t0, t1 = _dense_call(feat, outdeg, w1, w2[:F], w2[F:F + 1])
    a0, a1 = _agg_call(t0, t1, srca3, dstd3, zrows)
    return _final_call(a0, a1, indeg, v)


# speedup vs baseline: 1.5196x; 1.5196x over previous
import jax
import jax.numpy as jnp
from jax import lax
from jax.experimental import pallas as pl
from jax.experimental.pallas import tpu as pltpu
from jax.experimental.pallas import tpu_sc as plsc

N = 10000
E = 320000
F = 128
R = 64
HF = 64
HQ = 32
CHUNK = 128
NSUB = 16
NB = 160
EPAD = NSUB * NB * CHUNK - E
IB = 32
NIB = NB // IB
ROWS_PT = N // NSUB
NA = N + 8
DH = 16
QSCALE = 64.0
QCLIP = -4.0

_mesh = plsc.VectorSubcoreMesh(core_axis_name="c", subcore_axis_name="s")
_sc_params = pltpu.CompilerParams(use_tc_tiling_on_sc=False)


def _deg_body(srcd3, dstd3, zhist, onesb, degs, degd, hist, idxb, onesv, sem):
    del sem
    c = lax.axis_index("c")
    s = lax.axis_index("s")
    row0 = s * ROWS_PT

    pltpu.sync_copy(zhist, hist.at[pl.ds(row0, ROWS_PT)])
    pltpu.sync_copy(onesb, onesv)
    plsc.subcore_barrier()

    def run(idx3, out):
        def outer(b, carry):
            pltpu.sync_copy(idx3.at[s, pl.ds(b * IB, IB)], idxb)

            def body(k, carry2):
                pltpu.sync_copy(onesv, hist.at[idxb.at[k]], add=True)
                return carry2

            return lax.fori_loop(0, IB, body, carry)

        lax.fori_loop(0, NIB, outer, 0)
        plsc.subcore_barrier()
        pltpu.sync_copy(hist.at[pl.ds(row0, ROWS_PT)],
                        out.at[pl.ds(row0, ROWS_PT)])

    pl.when(c == 0)(lambda: run(srcd3, degs))
    pl.when(c == 1)(lambda: run(dstd3, degd))


_deg_call = pl.kernel(
    _deg_body,
    out_type=[jax.ShapeDtypeStruct((N, DH), jnp.float32),
              jax.ShapeDtypeStruct((N, DH), jnp.float32)],
    mesh=_mesh,
    scratch_types=[
        pltpu.VMEM_SHARED((NA, DH), jnp.float32),
        pltpu.VMEM((IB, CHUNK), jnp.int32),
        pltpu.VMEM((CHUNK, DH), jnp.float32),
        pltpu.SemaphoreType.DMA,
    ],
    compiler_params=_sc_params,
)


def _dense_body(feat_ref, deg_ref, w1_ref, w2a_ref, w2b_ref,
                ts0_ref, ts1_ref, tq0_ref, tq1_ref):
    x = feat_ref[...] * lax.rsqrt(jnp.maximum(deg_ref[...], 1.0))
    sfull = jnp.dot(x, w1_ref[...], preferred_element_type=jnp.float32)
    z = jnp.dot(x, w2a_ref[...], preferred_element_type=jnp.float32) + w2b_ref[...]
    t = jnp.tanh(z)
    lp = jnp.maximum(jnp.log(jnp.maximum(jnp.abs(t), 1e-30)), QCLIP)
    qm = jnp.round(lp * QSCALE).astype(jnp.int32)
    q = (2 * qm + (t < 0).astype(jnp.int32)).astype(jnp.int16)
    ts0_ref[...] = sfull[:, :HF]
    ts1_ref[...] = sfull[:, HF:]
    tq0_ref[...] = q[:, :HQ]
    tq1_ref[...] = q[:, HQ:]


_BLK = 1000

_dense_call = pl.pallas_call(
    _dense_body,
    grid=(N // _BLK,),
    in_specs=[
        pl.BlockSpec((_BLK, F), lambda i: (i, 0)),
        pl.BlockSpec((_BLK, 1), lambda i: (i, 0)),
        pl.BlockSpec((F, F), lambda i: (0, 0)),
        pl.BlockSpec((F, R), lambda i: (0, 0)),
        pl.BlockSpec((1, R), lambda i: (0, 0)),
    ],
    out_specs=[
        pl.BlockSpec((_BLK, HF), lambda i: (i, 0)),
        pl.BlockSpec((_BLK, HF), lambda i: (i, 0)),
        pl.BlockSpec((_BLK, HQ), lambda i: (i, 0)),
        pl.BlockSpec((_BLK, HQ), lambda i: (i, 0)),
    ],
    out_shape=[jax.ShapeDtypeStruct((N, HF), jnp.float32),
               jax.ShapeDtypeStruct((N, HF), jnp.float32),
               jax.ShapeDtypeStruct((N, HQ), jnp.int16),
               jax.ShapeDtypeStruct((N, HQ), jnp.int16)],
)


def _agg_body(ts0, ts1, tq0, tq1, srca3, dsta3, zs, zq, a0, a1, aq0, aq1,
              accs, accq, sidxb, didxb, rs0, rs1, rq0, rq1,
              gs0, gs1, ss0, ss1):
    c = lax.axis_index("c")
    s = lax.axis_index("s")
    row0 = s * ROWS_PT

    pltpu.sync_copy(zs, accs.at[pl.ds(row0, ROWS_PT)])
    pltpu.sync_copy(zq, accq.at[pl.ds(row0, ROWS_PT)])
    plsc.subcore_barrier()

    def run(ts, tq, outs, outq):
        def outer(b, carry):
            pltpu.sync_copy(srca3.at[s, pl.ds(b * IB, IB)], sidxb)
            pltpu.sync_copy(dsta3.at[s, pl.ds(b * IB, IB)], didxb)
            pltpu.async_copy(ts.at[sidxb.at[0]], rs0, gs0)
            pltpu.async_copy(tq.at[sidxb.at[0]], rq0, gs0)

            def body(i, carry2):
                k0 = 2 * i
                k1 = k0 + 1
                pltpu.async_copy(ts.at[sidxb.at[k1]], rs1, gs1)
                pltpu.async_copy(tq.at[sidxb.at[k1]], rq1, gs1)
                pltpu.make_async_copy(ts.at[sidxb.at[k0]], rs0, gs0).wait()
                pltpu.make_async_copy(tq.at[sidxb.at[k0]], rq0, gs0).wait()
                pltpu.async_copy(rs0, accs.at[didxb.at[k0]], ss0, add=True)
                pltpu.async_copy(rq0, accq.at[didxb.at[k0]], ss0, add=True)
                pltpu.make_async_copy(rs0, accs.at[didxb.at[k0]], ss0).wait()
                pltpu.make_async_copy(rq0, accq.at[didxb.at[k0]], ss0).wait()

                def _prefetch0():
                    pltpu.async_copy(ts.at[sidxb.at[k0 + 2]], rs0, gs0)
                    pltpu.async_copy(tq.at[sidxb.at[k0 + 2]], rq0, gs0)

                pl.when(i < IB // 2 - 1)(_prefetch0)
                pltpu.make_async_copy(ts.at[sidxb.at[k1]], rs1, gs1).wait()
                pltpu.make_async_copy(tq.at[sidxb.at[k1]], rq1, gs1).wait()
                pltpu.async_copy(rs1, accs.at[didxb.at[k1]], ss1, add=True)
                pltpu.async_copy(rq1, accq.at[didxb.at[k1]], ss1, add=True)
                pltpu.make_async_copy(rs1, accs.at[didxb.at[k1]], ss1).wait()
                pltpu.make_async_copy(rq1, accq.at[didxb.at[k1]], ss1).wait()
                return carry2

            return lax.fori_loop(0, IB // 2, body, carry)

        lax.fori_loop(0, NIB, outer, 0)
        plsc.subcore_barrier()
        pltpu.sync_copy(accs.at[pl.ds(row0, ROWS_PT)],
                        outs.at[pl.ds(row0, ROWS_PT)])
        pltpu.sync_copy(accq.at[pl.ds(row0, ROWS_PT)],
                        outq.at[pl.ds(row0, ROWS_PT)])

    pl.when(c == 0)(lambda: run(ts0, tq0, a0, aq0))
    pl.when(c == 1)(lambda: run(ts1, tq1, a1, aq1))


_agg_call = pl.kernel(
    _agg_body,
    out_type=[jax.ShapeDtypeStruct((N, HF), jnp.float32),
              jax.ShapeDtypeStruct((N, HF), jnp.float32),
              jax.ShapeDtypeStruct((N, HQ), jnp.int16),
              jax.ShapeDtypeStruct((N, HQ), jnp.int16)],
    mesh=_mesh,
    scratch_types=[
        pltpu.VMEM_SHARED((NA, HF), jnp.float32),
        pltpu.VMEM_SHARED((NA, HQ), jnp.int16),
        pltpu.VMEM((IB, CHUNK), jnp.int32),
        pltpu.VMEM((IB, CHUNK), jnp.int32),
        pltpu.VMEM((CHUNK, HF), jnp.float32),
        pltpu.VMEM((CHUNK, HF), jnp.float32),
        pltpu.VMEM((CHUNK, HQ), jnp.int16),
        pltpu.VMEM((CHUNK, HQ), jnp.int16),
        pltpu.SemaphoreType.DMA,
        pltpu.SemaphoreType.DMA,
        pltpu.SemaphoreType.DMA,
        pltpu.SemaphoreType.DMA,
    ],
    compiler_params=_sc_params,
)


def _final_body(a0_ref, a1_ref, q0_ref, q1_ref, deg_ref, v_ref, out_ref):
    indeg = deg_ref[...]
    h_sum = jnp.concatenate([a0_ref[...], a1_ref[...]], axis=1)
    qs = jnp.concatenate([q0_ref[...], q1_ref[...]], axis=1).astype(jnp.int32)
    par = jnp.bitwise_and(qs, 1)
    sign = (1 - 2 * par).astype(jnp.float32)
    lp = ((qs - par) // 2).astype(jnp.float32) * (1.0 / QSCALE)
    h_prod = sign * jnp.exp(lp) * (indeg > 0).astype(jnp.float32)
    r = h_sum + jnp.dot(h_prod, v_ref[...], preferred_element_type=jnp.float32)
    out_ref[...] = r * lax.rsqrt(jnp.maximum(indeg, 1.0))


_final_call = pl.pallas_call(
    _final_body,
    grid=(N // _BLK,),
    in_specs=[
        pl.BlockSpec((_BLK, HF), lambda i: (i, 0)),
        pl.BlockSpec((_BLK, HF), lambda i: (i, 0)),
        pl.BlockSpec((_BLK, HQ), lambda i: (i, 0)),
        pl.BlockSpec((_BLK, HQ), lambda i: (i, 0)),
        pl.BlockSpec((_BLK, 1), lambda i: (i, 0)),
        pl.BlockSpec((R, F), lambda i: (0, 0)),
    ],
    out_specs=pl.BlockSpec((_BLK, F), lambda i: (i, 0)),
    out_shape=jax.ShapeDtypeStruct((N, F), jnp.float32),
)


def kernel(feat, edge_index, w1, w2, v):
    src = edge_index[0]
    dst = edge_index[1]
    padn = jnp.full((EPAD,), N, jnp.int32)
    srcd3 = jnp.concatenate([src, padn]).reshape(NSUB, NB, CHUNK)
    dstd3 = jnp.concatenate([dst, padn]).reshape(NSUB, NB, CHUNK)
    srca3 = jnp.concatenate(
        [src, jnp.zeros((EPAD,), jnp.int32)]).reshape(NSUB, NB, CHUNK)
    zhist = jnp.zeros((ROWS_PT, DH), jnp.float32)
    onesb = jnp.concatenate(
        [jnp.ones((CHUNK, 1), jnp.float32),
         jnp.zeros((CHUNK, DH - 1), jnp.float32)], axis=1)
    zs = jnp.zeros((ROWS_PT, HF), jnp.float32)
    zq = jnp.zeros((ROWS_PT, HQ), jnp.int16)

    degs, degd = _deg_call(srcd3, dstd3, zhist, onesb)
    outdeg = degs[:, 0:1]
    indeg = degd[:, 0:1]
    ts0, ts1, tq0, tq1 = _dense_call(feat, outdeg, w1, w2[:F], w2[F:F + 1])
    a0, a1, aq0, aq1 = _agg_call(ts0, ts1, tq0, tq1, srca3, dstd3, zs, zq)
    return _final_call(a0, a1, aq0, aq1, indeg, v)


# speedup vs baseline: 1.5986x; 1.0520x over previous
import jax
import jax.numpy as jnp
from jax import lax
from jax.experimental import pallas as pl
from jax.experimental.pallas import tpu as pltpu
from jax.experimental.pallas import tpu_sc as plsc

N = 10000
E = 320000
F = 128
R = 64
HF = 64
HQ = 32
CHUNK = 128
NSUB = 16
NB = 160
EPAD = NSUB * NB * CHUNK - E
IB = 32
NIB = NB // IB
IBA = 80
NIBA = NB // IBA
NBUF = 4
ROWS_PT = N // NSUB
NA = N + 8
DH = 8
QSCALE = 64.0
QCLIP = -4.0

_mesh = plsc.VectorSubcoreMesh(core_axis_name="c", subcore_axis_name="s")
_sc_params = pltpu.CompilerParams(use_tc_tiling_on_sc=False)


def _deg_body(srcd3, dstd3, zhist, onesb, degs, degd, hist, idxb, onesv, sem):
    del sem
    c = lax.axis_index("c")
    s = lax.axis_index("s")
    row0 = s * ROWS_PT

    pltpu.sync_copy(zhist, hist.at[pl.ds(row0, ROWS_PT)])
    pltpu.sync_copy(onesb, onesv)
    plsc.subcore_barrier()

    def run(idx3, out):
        def outer(b, carry):
            pltpu.sync_copy(idx3.at[s, pl.ds(b * IB, IB)], idxb)

            def body(k, carry2):
                pltpu.sync_copy(onesv, hist.at[idxb.at[k]], add=True)
                return carry2

            return lax.fori_loop(0, IB, body, carry)

        lax.fori_loop(0, NIB, outer, 0)
        plsc.subcore_barrier()
        pltpu.sync_copy(hist.at[pl.ds(row0, ROWS_PT)],
                        out.at[pl.ds(row0, ROWS_PT)])

    pl.when(c == 0)(lambda: run(srcd3, degs))
    pl.when(c == 1)(lambda: run(dstd3, degd))


_deg_call = pl.kernel(
    _deg_body,
    out_type=[jax.ShapeDtypeStruct((N, DH), jnp.float32),
              jax.ShapeDtypeStruct((N, DH), jnp.float32)],
    mesh=_mesh,
    scratch_types=[
        pltpu.VMEM_SHARED((NA, DH), jnp.float32),
        pltpu.VMEM((IB, CHUNK), jnp.int32),
        pltpu.VMEM((CHUNK, DH), jnp.float32),
        pltpu.SemaphoreType.DMA,
    ],
    compiler_params=_sc_params,
)


def _dense_body(feat_ref, deg_ref, w1_ref, w2a_ref, w2b_ref,
                ts0_ref, ts1_ref, tq0_ref, tq1_ref):
    x = feat_ref[...] * lax.rsqrt(jnp.maximum(deg_ref[...], 1.0))
    sfull = jnp.dot(x, w1_ref[...], preferred_element_type=jnp.float32)
    z = jnp.dot(x, w2a_ref[...], preferred_element_type=jnp.float32) + w2b_ref[...]
    t = jnp.tanh(z)
    lp = jnp.maximum(jnp.log(jnp.maximum(jnp.abs(t), 1e-30)), QCLIP)
    qm = jnp.round(lp * QSCALE).astype(jnp.int32)
    q = (2 * qm + (t < 0).astype(jnp.int32)).astype(jnp.int16)
    ts0_ref[...] = sfull[:, :HF]
    ts1_ref[...] = sfull[:, HF:]
    tq0_ref[...] = q[:, :HQ]
    tq1_ref[...] = q[:, HQ:]


_BLK = 1000

_dense_call = pl.pallas_call(
    _dense_body,
    grid=(N // _BLK,),
    in_specs=[
        pl.BlockSpec((_BLK, F), lambda i: (i, 0)),
        pl.BlockSpec((_BLK, 1), lambda i: (i, 0)),
        pl.BlockSpec((F, F), lambda i: (0, 0)),
        pl.BlockSpec((F, R), lambda i: (0, 0)),
        pl.BlockSpec((1, R), lambda i: (0, 0)),
    ],
    out_specs=[
        pl.BlockSpec((_BLK, HF), lambda i: (i, 0)),
        pl.BlockSpec((_BLK, HF), lambda i: (i, 0)),
        pl.BlockSpec((_BLK, HQ), lambda i: (i, 0)),
        pl.BlockSpec((_BLK, HQ), lambda i: (i, 0)),
    ],
    out_shape=[jax.ShapeDtypeStruct((N, HF), jnp.float32),
               jax.ShapeDtypeStruct((N, HF), jnp.float32),
               jax.ShapeDtypeStruct((N, HQ), jnp.int16),
               jax.ShapeDtypeStruct((N, HQ), jnp.int16)],
)


def _agg_body(ts0, ts1, tq0, tq1, srca3, dsta3, zs, zq, a0, a1, aq0, aq1,
              accs, accq, sidxb, didxb, rs, rq, gsems, ssems):
    c = lax.axis_index("c")
    s = lax.axis_index("s")
    row0 = s * ROWS_PT

    pltpu.sync_copy(zs, accs.at[pl.ds(row0, ROWS_PT)])
    pltpu.sync_copy(zq, accq.at[pl.ds(row0, ROWS_PT)])
    plsc.subcore_barrier()

    def run(ts, tq, outs, outq):
        def outer(b, carry):
            pltpu.sync_copy(srca3.at[s, pl.ds(b * IBA, IBA)], sidxb)
            pltpu.sync_copy(dsta3.at[s, pl.ds(b * IBA, IBA)], didxb)
            for j in range(NBUF):
                pltpu.async_copy(ts.at[sidxb.at[j]], rs[j], gsems[j])
                pltpu.async_copy(tq.at[sidxb.at[j]], rq[j], gsems[j])

            def body(i, carry2):
                k = NBUF * i
                for j in range(NBUF):
                    pltpu.make_async_copy(
                        ts.at[sidxb.at[k + j]], rs[j], gsems[j]).wait()
                    pltpu.make_async_copy(
                        tq.at[sidxb.at[k + j]], rq[j], gsems[j]).wait()
                    pltpu.async_copy(
                        rs[j], accs.at[didxb.at[k + j]], ssems[j], add=True)
                    pltpu.async_copy(
                        rq[j], accq.at[didxb.at[k + j]], ssems[j], add=True)
                for j in range(NBUF):
                    pltpu.make_async_copy(
                        rs[j], accs.at[didxb.at[k + j]], ssems[j]).wait()
                    pltpu.make_async_copy(
                        rq[j], accq.at[didxb.at[k + j]], ssems[j]).wait()

                    def _prefetch(jj=j):
                        pltpu.async_copy(ts.at[sidxb.at[k + NBUF + jj]],
                                         rs[jj], gsems[jj])
                        pltpu.async_copy(tq.at[sidxb.at[k + NBUF + jj]],
                                         rq[jj], gsems[jj])

                    pl.when(i < IBA // NBUF - 1)(_prefetch)
                return carry2

            return lax.fori_loop(0, IBA // NBUF, body, carry)

        lax.fori_loop(0, NIBA, outer, 0)
        plsc.subcore_barrier()
        pltpu.sync_copy(accs.at[pl.ds(row0, ROWS_PT)],
                        outs.at[pl.ds(row0, ROWS_PT)])
        pltpu.sync_copy(accq.at[pl.ds(row0, ROWS_PT)],
                        outq.at[pl.ds(row0, ROWS_PT)])

    pl.when(c == 0)(lambda: run(ts0, tq0, a0, aq0))
    pl.when(c == 1)(lambda: run(ts1, tq1, a1, aq1))


_agg_call = pl.kernel(
    _agg_body,
    out_type=[jax.ShapeDtypeStruct((N, HF), jnp.float32),
              jax.ShapeDtypeStruct((N, HF), jnp.float32),
              jax.ShapeDtypeStruct((N, HQ), jnp.int16),
              jax.ShapeDtypeStruct((N, HQ), jnp.int16)],
    mesh=_mesh,
    scratch_types=[
        pltpu.VMEM_SHARED((NA, HF), jnp.float32),
        pltpu.VMEM_SHARED((NA, HQ), jnp.int16),
        pltpu.VMEM((IBA, CHUNK), jnp.int32),
        pltpu.VMEM((IBA, CHUNK), jnp.int32),
        [pltpu.VMEM((CHUNK, HF), jnp.float32)] * NBUF,
        [pltpu.VMEM((CHUNK, HQ), jnp.int16)] * NBUF,
        [pltpu.SemaphoreType.DMA] * NBUF,
        [pltpu.SemaphoreType.DMA] * NBUF,
    ],
    compiler_params=_sc_params,
)


def _final_body(a0_ref, a1_ref, q0_ref, q1_ref, deg_ref, v_ref, out_ref):
    indeg = deg_ref[...]
    h_sum = jnp.concatenate([a0_ref[...], a1_ref[...]], axis=1)
    qs = jnp.concatenate([q0_ref[...], q1_ref[...]], axis=1).astype(jnp.int32)
    par = jnp.bitwise_and(qs, 1)
    sign = (1 - 2 * par).astype(jnp.float32)
    lp = ((qs - par) // 2).astype(jnp.float32) * (1.0 / QSCALE)
    h_prod = sign * jnp.exp(lp) * (indeg > 0).astype(jnp.float32)
    r = h_sum + jnp.dot(h_prod, v_ref[...], preferred_element_type=jnp.float32)
    out_ref[...] = r * lax.rsqrt(jnp.maximum(indeg, 1.0))


_final_call = pl.pallas_call(
    _final_body,
    grid=(N // _BLK,),
    in_specs=[
        pl.BlockSpec((_BLK, HF), lambda i: (i, 0)),
        pl.BlockSpec((_BLK, HF), lambda i: (i, 0)),
        pl.BlockSpec((_BLK, HQ), lambda i: (i, 0)),
        pl.BlockSpec((_BLK, HQ), lambda i: (i, 0)),
        pl.BlockSpec((_BLK, 1), lambda i: (i, 0)),
        pl.BlockSpec((R, F), lambda i: (0, 0)),
    ],
    out_specs=pl.BlockSpec((_BLK, F), lambda i: (i, 0)),
    out_shape=jax.ShapeDtypeStruct((N, F), jnp.float32),
)


def kernel(feat, edge_index, w1, w2, v):
    src = edge_index[0]
    dst = edge_index[1]
    padn = jnp.full((EPAD,), N, jnp.int32)
    srcd3 = jnp.concatenate([src, padn]).reshape(NSUB, NB, CHUNK)
    dstd3 = jnp.concatenate([dst, padn]).reshape(NSUB, NB, CHUNK)
    srca3 = jnp.concatenate(
        [src, jnp.zeros((EPAD,), jnp.int32)]).reshape(NSUB, NB, CHUNK)
    zhist = jnp.zeros((ROWS_PT, DH), jnp.float32)
    onesb = jnp.concatenate(
        [jnp.ones((CHUNK, 1), jnp.float32),
         jnp.zeros((CHUNK, DH - 1), jnp.float32)], axis=1)
    zs = jnp.zeros((ROWS_PT, HF), jnp.float32)
    zq = jnp.zeros((ROWS_PT, HQ), jnp.int16)

    degs, degd = _deg_call(srcd3, dstd3, zhist, onesb)
    outdeg = degs[:, 0:1]
    indeg = degd[:, 0:1]
    ts0, ts1, tq0, tq1 = _dense_call(feat, outdeg, w1, w2[:F], w2[F:F + 1])
    a0, a1, aq0, aq1 = _agg_call(ts0, ts1, tq0, tq1, srca3, dstd3, zs, zq)
    return _final_call(a0, a1, aq0, aq1, indeg, v)


# speedup vs baseline: 2.1668x; 1.3554x over previous
import jax
import jax.numpy as jnp
from jax import lax
from jax.experimental import pallas as pl
from jax.experimental.pallas import tpu as pltpu
from jax.experimental.pallas import tpu_sc as plsc

N = 10000
E = 320000
F = 128
R = 64
HF = 64
HQ = 32
CHUNK = 128
NSUB = 16
NB = 160
EPAD = NSUB * NB * CHUNK - E
IB = 32
NIB = NB // IB
IBA = 80
NIBA = NB // IBA
NBUF = 4
ROWS_PT = N // NSUB
NA = N + 8
DH = 8
QSCALE = 64.0
QCLIP = -4.0

_mesh = plsc.VectorSubcoreMesh(core_axis_name="c", subcore_axis_name="s")
_sc_params = pltpu.CompilerParams(use_tc_tiling_on_sc=False)


def _deg_body(srcd3, dstd3, zhist, onesb, degs, degd, hist, idxb, onesv, sem):
    del sem
    c = lax.axis_index("c")
    s = lax.axis_index("s")
    row0 = s * ROWS_PT

    pltpu.sync_copy(zhist, hist.at[pl.ds(row0, ROWS_PT)])
    pltpu.sync_copy(onesb, onesv)
    plsc.subcore_barrier()

    def run(idx3, out):
        def outer(b, carry):
            pltpu.sync_copy(idx3.at[s, pl.ds(b * IB, IB)], idxb)

            def body(k, carry2):
                pltpu.sync_copy(onesv, hist.at[idxb.at[k]], add=True)
                return carry2

            return lax.fori_loop(0, IB, body, carry)

        lax.fori_loop(0, NIB, outer, 0)
        plsc.subcore_barrier()
        pltpu.sync_copy(hist.at[pl.ds(row0, ROWS_PT)],
                        out.at[pl.ds(row0, ROWS_PT)])

    pl.when(c == 0)(lambda: run(srcd3, degs))
    pl.when(c == 1)(lambda: run(dstd3, degd))


_deg_call = pl.kernel(
    _deg_body,
    out_type=[jax.ShapeDtypeStruct((N, DH), jnp.float32),
              jax.ShapeDtypeStruct((N, DH), jnp.float32)],
    mesh=_mesh,
    scratch_types=[
        pltpu.VMEM_SHARED((NA, DH), jnp.float32),
        pltpu.VMEM((IB, CHUNK), jnp.int32),
        pltpu.VMEM((CHUNK, DH), jnp.float32),
        pltpu.SemaphoreType.DMA,
    ],
    compiler_params=_sc_params,
)


def _dense_body(feat_ref, deg_ref, w1_ref, w2a_ref, w2b_ref,
                ts0_ref, ts1_ref, tq0_ref, tq1_ref):
    x = feat_ref[...] * lax.rsqrt(jnp.maximum(deg_ref[...], 1.0))
    sfull = jnp.dot(x, w1_ref[...], preferred_element_type=jnp.float32)
    z = jnp.dot(x, w2a_ref[...], preferred_element_type=jnp.float32) + w2b_ref[...]
    t = jnp.tanh(z)
    lp = jnp.maximum(jnp.log(jnp.maximum(jnp.abs(t), 1e-30)), QCLIP)
    qm = jnp.round(lp * QSCALE).astype(jnp.int32)
    q = (2 * qm + (t < 0).astype(jnp.int32)).astype(jnp.int16)
    ts0_ref[...] = sfull[:, :HF].astype(jnp.bfloat16)
    ts1_ref[...] = sfull[:, HF:].astype(jnp.bfloat16)
    tq0_ref[...] = q[:, :HQ]
    tq1_ref[...] = q[:, HQ:]


_BLK = 1000

_dense_call = pl.pallas_call(
    _dense_body,
    grid=(N // _BLK,),
    in_specs=[
        pl.BlockSpec((_BLK, F), lambda i: (i, 0)),
        pl.BlockSpec((_BLK, 1), lambda i: (i, 0)),
        pl.BlockSpec((F, F), lambda i: (0, 0)),
        pl.BlockSpec((F, R), lambda i: (0, 0)),
        pl.BlockSpec((1, R), lambda i: (0, 0)),
    ],
    out_specs=[
        pl.BlockSpec((_BLK, HF), lambda i: (i, 0)),
        pl.BlockSpec((_BLK, HF), lambda i: (i, 0)),
        pl.BlockSpec((_BLK, HQ), lambda i: (i, 0)),
        pl.BlockSpec((_BLK, HQ), lambda i: (i, 0)),
    ],
    out_shape=[jax.ShapeDtypeStruct((N, HF), jnp.bfloat16),
               jax.ShapeDtypeStruct((N, HF), jnp.bfloat16),
               jax.ShapeDtypeStruct((N, HQ), jnp.int16),
               jax.ShapeDtypeStruct((N, HQ), jnp.int16)],
)


def _agg_body(ts0, ts1, tq0, tq1, srca3, dsta3, zs, zq, a0, a1, aq0, aq1,
              accs, accq, sidxb, didxb, rs, rq, gsems, ssems):
    c = lax.axis_index("c")
    s = lax.axis_index("s")
    row0 = s * ROWS_PT

    pltpu.sync_copy(zs, accs.at[pl.ds(row0, ROWS_PT)])
    pltpu.sync_copy(zq, accq.at[pl.ds(row0, ROWS_PT)])
    plsc.subcore_barrier()

    def run(ts, tq, outs, outq):
        def outer(b, carry):
            pltpu.sync_copy(srca3.at[s, pl.ds(b * IBA, IBA)], sidxb)
            pltpu.sync_copy(dsta3.at[s, pl.ds(b * IBA, IBA)], didxb)
            for j in range(NBUF):
                pltpu.async_copy(ts.at[sidxb.at[j]], rs[j], gsems[j])
                pltpu.async_copy(tq.at[sidxb.at[j]], rq[j], gsems[j])

            def body(i, carry2):
                k = NBUF * i
                for j in range(NBUF):
                    pltpu.make_async_copy(
                        ts.at[sidxb.at[k + j]], rs[j], gsems[j]).wait()
                    pltpu.make_async_copy(
                        tq.at[sidxb.at[k + j]], rq[j], gsems[j]).wait()
                    pltpu.async_copy(
                        rs[j], accs.at[didxb.at[k + j]], ssems[j], add=True)
                    pltpu.async_copy(
                        rq[j], accq.at[didxb.at[k + j]], ssems[j], add=True)
                for j in range(NBUF):
                    pltpu.make_async_copy(
                        rs[j], accs.at[didxb.at[k + j]], ssems[j]).wait()
                    pltpu.make_async_copy(
                        rq[j], accq.at[didxb.at[k + j]], ssems[j]).wait()

                    def _prefetch(jj=j):
                        pltpu.async_copy(ts.at[sidxb.at[k + NBUF + jj]],
                                         rs[jj], gsems[jj])
                        pltpu.async_copy(tq.at[sidxb.at[k + NBUF + jj]],
                                         rq[jj], gsems[jj])

                    pl.when(i < IBA // NBUF - 1)(_prefetch)
                return carry2

            return lax.fori_loop(0, IBA // NBUF, body, carry)

        lax.fori_loop(0, NIBA, outer, 0)
        plsc.subcore_barrier()
        pltpu.sync_copy(accs.at[pl.ds(row0, ROWS_PT)],
                        outs.at[pl.ds(row0, ROWS_PT)])
        pltpu.sync_copy(accq.at[pl.ds(row0, ROWS_PT)],
                        outq.at[pl.ds(row0, ROWS_PT)])

    pl.when(c == 0)(lambda: run(ts0, tq0, a0, aq0))
    pl.when(c == 1)(lambda: run(ts1, tq1, a1, aq1))


_agg_call = pl.kernel(
    _agg_body,
    out_type=[jax.ShapeDtypeStruct((N, HF), jnp.bfloat16),
              jax.ShapeDtypeStruct((N, HF), jnp.bfloat16),
              jax.ShapeDtypeStruct((N, HQ), jnp.int16),
              jax.ShapeDtypeStruct((N, HQ), jnp.int16)],
    mesh=_mesh,
    scratch_types=[
        pltpu.VMEM_SHARED((NA, HF), jnp.bfloat16),
        pltpu.VMEM_SHARED((NA, HQ), jnp.int16),
        pltpu.VMEM((IBA, CHUNK), jnp.int32),
        pltpu.VMEM((IBA, CHUNK), jnp.int32),
        [pltpu.VMEM((CHUNK, HF), jnp.bfloat16)] * NBUF,
        [pltpu.VMEM((CHUNK, HQ), jnp.int16)] * NBUF,
        [pltpu.SemaphoreType.DMA] * NBUF,
        [pltpu.SemaphoreType.DMA] * NBUF,
    ],
    compiler_params=_sc_params,
)


def _final_body(a0_ref, a1_ref, q0_ref, q1_ref, deg_ref, v_ref, out_ref):
    indeg = deg_ref[...]
    h_sum = jnp.concatenate([a0_ref[...], a1_ref[...]],
                            axis=1).astype(jnp.float32)
    qs = jnp.concatenate([q0_ref[...], q1_ref[...]], axis=1).astype(jnp.int32)
    par = jnp.bitwise_and(qs, 1)
    sign = (1 - 2 * par).astype(jnp.float32)
    lp = ((qs - par) // 2).astype(jnp.float32) * (1.0 / QSCALE)
    h_prod = sign * jnp.exp(lp) * (indeg > 0).astype(jnp.float32)
    r = h_sum + jnp.dot(h_prod, v_ref[...], preferred_element_type=jnp.float32)
    out_ref[...] = r * lax.rsqrt(jnp.maximum(indeg, 1.0))


_final_call = pl.pallas_call(
    _final_body,
    grid=(N // _BLK,),
    in_specs=[
        pl.BlockSpec((_BLK, HF), lambda i: (i, 0)),
        pl.BlockSpec((_BLK, HF), lambda i: (i, 0)),
        pl.BlockSpec((_BLK, HQ), lambda i: (i, 0)),
        pl.BlockSpec((_BLK, HQ), lambda i: (i, 0)),
        pl.BlockSpec((_BLK, 1), lambda i: (i, 0)),
        pl.BlockSpec((R, F), lambda i: (0, 0)),
    ],
    out_specs=pl.BlockSpec((_BLK, F), lambda i: (i, 0)),
    out_shape=jax.ShapeDtypeStruct((N, F), jnp.float32),
)


def kernel(feat, edge_index, w1, w2, v):
    src = edge_index[0]
    dst = edge_index[1]
    padn = jnp.full((EPAD,), N, jnp.int32)
    srcd3 = jnp.concatenate([src, padn]).reshape(NSUB, NB, CHUNK)
    dstd3 = jnp.concatenate([dst, padn]).reshape(NSUB, NB, CHUNK)
    srca3 = jnp.concatenate(
        [src, jnp.zeros((EPAD,), jnp.int32)]).reshape(NSUB, NB, CHUNK)
    zhist = jnp.zeros((ROWS_PT, DH), jnp.float32)
    onesb = jnp.concatenate(
        [jnp.ones((CHUNK, 1), jnp.float32),
         jnp.zeros((CHUNK, DH - 1), jnp.float32)], axis=1)
    zs = jnp.zeros((ROWS_PT, HF), jnp.bfloat16)
    zq = jnp.zeros((ROWS_PT, HQ), jnp.int16)

    degs, degd = _deg_call(srcd3, dstd3, zhist, onesb)
    outdeg = degs[:, 0:1]
    indeg = degd[:, 0:1]
    ts0, ts1, tq0, tq1 = _dense_call(feat, outdeg, w1, w2[:F], w2[F:F + 1])
    a0, a1, aq0, aq1 = _agg_call(ts0, ts1, tq0, tq1, srca3, dstd3, zs, zq)
    return _final_call(a0, a1, aq0, aq1, indeg, v)
